# baseline (speedup 1.0000x reference)
import jax
import jax.numpy as jnp
from jax.experimental import pallas as pl
from jax.experimental.pallas import tpu as pltpu

_B, _C, _N = 4, 3, 4096
_M = 4096
_SL = 8
_NS = _M // _SL
_EPS = 1e-8


def _chamfer_batch(p_ref, g_ref, out_ref, facc_ref, bacc_ref):
    b = pl.program_id(0)

    @pl.when(b == 0)
    def _init_all():
        facc_ref[...] = jnp.zeros((1, 1), jnp.float32)
        bacc_ref[...] = jnp.zeros((1, 1), jnp.float32)

    g0 = g_ref[0, 0, :].reshape(1, _N)
    g1 = g_ref[0, 1, :].reshape(1, _N)
    g2 = g_ref[0, 2, :].reshape(1, _N)
    ng2 = g0 * g0 + g1 * g1 + g2 * g2

    def slab(i, carry):
        colmin, fsum = carry
        psl = p_ref[0, pl.ds(i * _SL, _SL), :]
        p0 = psl[:, 0].reshape(_SL, 1)
        p1 = psl[:, 1].reshape(_SL, 1)
        p2 = psl[:, 2].reshape(_SL, 1)
        np2 = p0 * p0 + p1 * p1 + p2 * p2
        t = (-2.0 * p0) * g0 + (-2.0 * p1) * g1 + (-2.0 * p2) * g2
        e = t + ng2
        f = t + np2
        rmin = jnp.min(e, axis=1, keepdims=True)
        fsum = fsum + jnp.sqrt(rmin + np2 + _EPS)
        colmin = jnp.minimum(colmin, f)
        return colmin, fsum

    colmin0 = jnp.full((_SL, _N), jnp.inf, jnp.float32)
    fsum0 = jnp.zeros((_SL, 1), jnp.float32)
    colmin, fsum = jax.lax.fori_loop(0, _NS, slab, (colmin0, fsum0))

    cmin = jnp.min(colmin, axis=0, keepdims=True) + ng2
    facc_ref[...] += jnp.sum(fsum, axis=(0, 1), keepdims=True)
    bacc_ref[...] += jnp.sum(jnp.sqrt(cmin + _EPS), axis=(0, 1), keepdims=True)

    @pl.when(b == _B - 1)
    def _emit():
        out_ref[...] = facc_ref[...] / (_B * _M) + bacc_ref[...] / (_B * _N)


def kernel(predict_pc, gt_pc):
    p_t = jnp.transpose(predict_pc, (0, 2, 1))
    out = pl.pallas_call(
        _chamfer_batch,
        grid=(_B,),
        in_specs=[
            pl.BlockSpec((1, _M, _C), lambda b: (b, 0, 0)),
            pl.BlockSpec((1, _C, _N), lambda b: (b, 0, 0)),
        ],
        out_specs=pl.BlockSpec((1, 1), lambda b: (0, 0)),
        out_shape=jax.ShapeDtypeStruct((1, 1), jnp.float32),
        scratch_shapes=[
            pltpu.VMEM((1, 1), jnp.float32),
            pltpu.VMEM((1, 1), jnp.float32),
        ],
    )(p_t, gt_pc)
    return out[0, 0]


# speedup vs baseline: 3.4966x; 3.4966x over previous
import jax
import jax.numpy as jnp
from jax.experimental import pallas as pl
from jax.experimental.pallas import tpu as pltpu

_B, _C, _M = 4, 3, 4096
_N = 4096
_MB = 1024
_MI = _M // _MB
_EPS = 1e-8


def _chamfer_tile(p_ref, g_ref, out_ref, colmin_ref, facc_ref, bacc_ref):
    b = pl.program_id(0)
    mi = pl.program_id(1)

    @pl.when(mi == 0)
    def _init_batch():
        colmin_ref[...] = jnp.full((1, _N), jnp.inf, jnp.float32)

    @pl.when((b == 0) & (mi == 0))
    def _init_all():
        facc_ref[...] = jnp.zeros((1, 1), jnp.float32)
        bacc_ref[...] = jnp.zeros((1, 1), jnp.float32)

    p = p_ref[0]
    g = g_ref[0]
    p0 = p[0, :].reshape(_MB, 1)
    p1 = p[1, :].reshape(_MB, 1)
    p2 = p[2, :].reshape(_MB, 1)
    g0 = g[0, :].reshape(1, _N)
    g1 = g[1, :].reshape(1, _N)
    g2 = g[2, :].reshape(1, _N)
    q0 = -2.0 * p0
    q1 = -2.0 * p1
    q2 = -2.0 * p2
    np2 = p0 * p0 + p1 * p1 + p2 * p2
    ng2 = g0 * g0 + g1 * g1 + g2 * g2
    t = q0 * g0 + q1 * g1 + q2 * g2
    e = t + ng2
    f = t + np2

    rowmin = np2 + jnp.min(e, axis=1, keepdims=True)
    facc_ref[...] += jnp.sum(jnp.sqrt(rowmin + _EPS), axis=(0, 1), keepdims=True)

    colmin_ref[...] = jnp.minimum(colmin_ref[...], jnp.min(f, axis=0, keepdims=True))

    @pl.when(mi == _MI - 1)
    def _finish_batch():
        bmin = ng2 + colmin_ref[...]
        bacc_ref[...] += jnp.sum(jnp.sqrt(bmin + _EPS), axis=(0, 1), keepdims=True)

    @pl.when((b == _B - 1) & (mi == _MI - 1))
    def _emit():
        out_ref[...] = facc_ref[...] / (_B * _M) + bacc_ref[...] / (_B * _N)


def kernel(predict_pc, gt_pc):
    out = pl.pallas_call(
        _chamfer_tile,
        grid=(_B, _MI),
        in_specs=[
            pl.BlockSpec((1, _C, _MB), lambda b, mi: (b, 0, mi)),
            pl.BlockSpec((1, _C, _N), lambda b, mi: (b, 0, 0)),
        ],
        out_specs=pl.BlockSpec((1, 1), lambda b, mi: (0, 0)),
        out_shape=jax.ShapeDtypeStruct((1, 1), jnp.float32),
        scratch_shapes=[
            pltpu.VMEM((1, _N), jnp.float32),
            pltpu.VMEM((1, 1), jnp.float32),
            pltpu.VMEM((1, 1), jnp.float32),
        ],
    )(predict_pc, gt_pc)
    return out[0, 0]
